# baseline (speedup 1.0000x reference)
import functools

import jax
import jax.numpy as jnp
from jax import lax
from jax.experimental import pallas as pl
from jax.experimental.pallas import tpu as pltpu
from jax.experimental.pallas import tpu_sc as plsc

N = 10000
E = 320000
D_IN = 128
D_H = 256
D_OUT = 128
D_PRED = 512
N_KINDS = 4

NC = 2
NS = 16
NW = NC * NS
L = 16

CH = 128
SG = 16
NCH_E = 80
EPAD = NW * NCH_E * CH
NCH1 = 2 * NCH_E
NPAD = 10240
RPT = NPAD // NS
REP = 8
DEGBASE = N + L


def _sc_agg_body(nchunk, table, px, zeros, out,
                 px_v, src_c, dst_c, rows_v, gsem, acc_sh):
    c = lax.axis_index("c")
    s = lax.axis_index("s")
    w = c * NS + s

    pltpu.sync_copy(zeros.at[pl.ds(s * RPT, RPT)], acc_sh.at[pl.ds(s * RPT, RPT)])
    pltpu.sync_copy(px.at[w], px_v)
    plsc.subcore_barrier()

    def chunk(j, carry):
        for t in range(CH // L):
            p16 = px_v[j, pl.ds(t * L, L)]
            src_c[pl.ds(t * L, L)] = p16 & 16383
            dst_c[pl.ds(t * L, L)] = p16 >> 14
        pltpu.async_copy(table.at[src_c], rows_v, gsem).wait()
        pltpu.sync_copy(rows_v, acc_sh.at[dst_c], add=True)
        return carry

    lax.fori_loop(0, nchunk, chunk, 0)
    plsc.subcore_barrier()

    pltpu.sync_copy(acc_sh.at[pl.ds(s * RPT, RPT)], out.at[c, pl.ds(s * RPT, RPT)])


def _make_sc_agg(nchunk):
    mesh = plsc.VectorSubcoreMesh(core_axis_name="c", subcore_axis_name="s")
    return pl.kernel(
        functools.partial(_sc_agg_body, nchunk),
        out_type=jax.ShapeDtypeStruct((NC, NPAD, D_IN), jnp.float32),
        mesh=mesh,
        scratch_types=[
            pltpu.VMEM((nchunk, CH), jnp.int32),
            pltpu.VMEM((CH,), jnp.int32),
            pltpu.VMEM((CH,), jnp.int32),
            pltpu.VMEM((CH, D_IN), jnp.float32),
            pltpu.SemaphoreType.DMA,
            pltpu.VMEM_SHARED((NPAD, D_IN), jnp.float32),
        ],
    )


_sc_agg1 = _make_sc_agg(NCH1)
_sc_agg2 = _make_sc_agg(NCH_E)


BLK = 400
NBLK = N // BLK
DBLK = NPAD // 8

_dot = functools.partial(jnp.dot, preferred_element_type=jnp.float32,
                         precision=lax.Precision.HIGHEST)


def _ext_body(s3, d3, rp, ep):
    sv = s3[...]
    dv = d3[...]
    rp[...] = sv | (dv << 14)
    ep[...] = (((dv & (REP * CH - 1)) + N)
               | (((dv >> 7) + DEGBASE) << 14))


def _tc_ext(src3, dst3):
    return pl.pallas_call(
        _ext_body,
        out_shape=[
            jax.ShapeDtypeStruct((NW, NCH_E, CH), jnp.int32),
            jax.ShapeDtypeStruct((NW, NCH_E, CH), jnp.int32),
        ],
    )(src3, dst3)


def _stage_a_body(x, nk, emb, h0):
    h = x[...]
    k = nk[...].astype(jnp.float32)
    for kk in range(N_KINDS):
        mask = jnp.where(k == kk, 1.0, 0.0)
        h = h + mask * emb[kk, :][None, :]
    h0[...] = h


def _tc_stage_a(x, nk2, emb):
    return pl.pallas_call(
        _stage_a_body,
        grid=(NBLK,),
        in_specs=[
            pl.BlockSpec((BLK, D_IN), lambda i: (i, 0)),
            pl.BlockSpec((BLK, 1), lambda i: (i, 0)),
            pl.BlockSpec((N_KINDS, D_IN), lambda i: (0, 0)),
        ],
        out_specs=pl.BlockSpec((BLK, D_IN), lambda i: (i, 0)),
        out_shape=jax.ShapeDtypeStruct((N, D_IN), jnp.float32),
    )(x, nk2, emb)


def _deg_body(degp, out):
    out[...] = jnp.maximum(jnp.sum(degp[...], axis=0), 1.0)[:, None]


def _tc_deg(degp):
    return pl.pallas_call(
        _deg_body,
        grid=(8,),
        in_specs=[pl.BlockSpec((NC, DBLK), lambda i: (0, i))],
        out_specs=pl.BlockSpec((DBLK, 1), lambda i: (i, 0)),
        out_shape=jax.ShapeDtypeStruct((NPAD, 1), jnp.float32),
    )(degp)


def _stage_b_body(h0, s0a, s0b, dg, w1s, w1n, b1, w2s, w2n, z, hs):
    deg = dg[...]
    a0 = (s0a[...] + s0b[...]) / deg
    h1 = jax.nn.relu(_dot(h0[...], w1s[...]) + _dot(a0, w1n[...]) + b1[...])
    z[...] = _dot(h1, w2n[...])
    hs[...] = _dot(h1, w2s[...])


def _tc_stage_b(h0, s0a, s0b, dg, W1s, W1n, b1, W2s, W2n):
    return pl.pallas_call(
        _stage_b_body,
        grid=(NBLK,),
        in_specs=[
            pl.BlockSpec((BLK, D_IN), lambda i: (i, 0)),
            pl.BlockSpec((BLK, D_IN), lambda i: (i, 0)),
            pl.BlockSpec((BLK, D_IN), lambda i: (i, 0)),
            pl.BlockSpec((BLK, 1), lambda i: (i, 0)),
            pl.BlockSpec((D_IN, D_H), lambda i: (0, 0)),
            pl.BlockSpec((D_IN, D_H), lambda i: (0, 0)),
            pl.BlockSpec((1, D_H), lambda i: (0, 0)),
            pl.BlockSpec((D_H, D_OUT), lambda i: (0, 0)),
            pl.BlockSpec((D_H, D_OUT), lambda i: (0, 0)),
        ],
        out_specs=[
            pl.BlockSpec((BLK, D_OUT), lambda i: (i, 0)),
            pl.BlockSpec((BLK, D_OUT), lambda i: (i, 0)),
        ],
        out_shape=[
            jax.ShapeDtypeStruct((N, D_OUT), jnp.float32),
            jax.ShapeDtypeStruct((N, D_OUT), jnp.float32),
        ],
    )(h0, s0a, s0b, dg, W1s, W1n, b1, W2s, W2n)


def _stage_c_body(hs, s1a, s1b, dg, b2, wp1, bp1, wp2, bp2, p):
    deg = dg[...]
    a1 = (s1a[...] + s1b[...]) / deg
    h2 = hs[...] + a1 + b2[...]
    g = jax.nn.relu(_dot(h2, wp1[...]) + bp1[...])
    p[...] = _dot(g, wp2[...]) + bp2[...]


def _tc_stage_c(hs, s1a, s1b, dg, b2, Wp1, bp1, Wp2, bp2):
    return pl.pallas_call(
        _stage_c_body,
        grid=(NBLK,),
        in_specs=[
            pl.BlockSpec((BLK, D_OUT), lambda i: (i, 0)),
            pl.BlockSpec((BLK, D_OUT), lambda i: (i, 0)),
            pl.BlockSpec((BLK, D_OUT), lambda i: (i, 0)),
            pl.BlockSpec((BLK, 1), lambda i: (i, 0)),
            pl.BlockSpec((1, D_OUT), lambda i: (0, 0)),
            pl.BlockSpec((D_OUT, D_PRED), lambda i: (0, 0)),
            pl.BlockSpec((1, D_PRED), lambda i: (0, 0)),
            pl.BlockSpec((D_PRED, D_OUT), lambda i: (0, 0)),
            pl.BlockSpec((1, D_OUT), lambda i: (0, 0)),
        ],
        out_specs=pl.BlockSpec((BLK, D_OUT), lambda i: (i, 0)),
        out_shape=jax.ShapeDtypeStruct((N, D_OUT), jnp.float32),
    )(hs, s1a, s1b, dg, b2, Wp1, bp1, Wp2, bp2)


def kernel(x, edge_index, node_kind, family_ids, kind_emb,
           W1s, W1n, b1, W2s, W2n, b2, Wp1, bp1, Wp2, bp2):
    src = edge_index[0]
    dst = edge_index[1]
    pad = EPAD - E
    src3 = jnp.concatenate([src, jnp.zeros((pad,), jnp.int32)]).reshape(NW, NCH_E, CH)
    dst3 = jnp.concatenate([dst, jnp.full((pad,), N, jnp.int32)]).reshape(NW, NCH_E, CH)
    rp, ep = _tc_ext(src3, dst3)
    px1 = jnp.stack([rp, ep], axis=2).reshape(NW, NCH1, CH)

    zeros = jnp.zeros((NPAD, D_IN), jnp.float32)
    eye_rep = jnp.tile(jnp.eye(CH, dtype=jnp.float32), (REP, 1))

    nk2 = node_kind[:, None]
    b1r = b1[None, :]
    b2r = b2[None, :]
    bp1r = bp1[None, :]
    bp2r = bp2[None, :]

    h0 = _tc_stage_a(x, nk2, kind_emb)
    table1 = jnp.concatenate([h0, eye_rep])
    s0 = _sc_agg1(table1, px1, zeros)
    dg = _tc_deg(s0[:, DEGBASE:DEGBASE + NPAD // CH, :].reshape(NC, NPAD))
    z, hs = _tc_stage_b(h0, s0[0], s0[1], dg, W1s, W1n, b1r, W2s, W2n)
    s1 = _sc_agg2(z, rp, zeros)
    p = _tc_stage_c(hs, s1[0], s1[1], dg, b2r, Wp1, bp1r, Wp2, bp2r)
    return p


# speedup vs baseline: 1.0704x; 1.0704x over previous
import functools

import jax
import jax.numpy as jnp
from jax import lax
from jax.experimental import pallas as pl
from jax.experimental.pallas import tpu as pltpu
from jax.experimental.pallas import tpu_sc as plsc

N = 10000
E = 320000
D_IN = 128
D_H = 256
D_OUT = 128
D_PRED = 512
N_KINDS = 4

NC = 2
NS = 16
NW = NC * NS
L = 16

CH = 128
SG = 16
NCH_E = 80
EPAD = NW * NCH_E * CH
NCH1 = 2 * NCH_E
NPAD = 10240
RPT = NPAD // NS
REP = 8
DEGBASE = N + L


PXR = 80


def _sc_agg_body(nchunk, table, px, zeros, out,
                 px_v, src_c, dst_c, rows_v, gsem_a, gsem_b, acc_sh):
    c = lax.axis_index("c")
    s = lax.axis_index("s")
    w = c * NS + s

    def unpack(j, b):
        r = j - jnp.where(j >= PXR, PXR, 0)
        for t in range(CH // L):
            p16 = px_v[r, pl.ds(t * L, L)]
            src_c[b, pl.ds(t * L, L)] = p16 & 16383
            dst_c[b, pl.ds(t * L, L)] = p16 >> 14

    def fire(b, sem):
        return pltpu.async_copy(table.at[src_c.at[b]], rows_v.at[b], sem)

    def wait(b, sem):
        pltpu.make_async_copy(table.at[src_c.at[b]], rows_v.at[b], sem).wait()

    def scat(b):
        pltpu.sync_copy(rows_v.at[b], acc_sh.at[dst_c.at[b]], add=True)

    pltpu.sync_copy(zeros.at[pl.ds(s * RPT, RPT)], acc_sh.at[pl.ds(s * RPT, RPT)])
    pltpu.sync_copy(px.at[w, pl.ds(0, PXR)], px_v)
    plsc.subcore_barrier()
    unpack(jnp.int32(0), 0)
    fire(0, gsem_a)

    def pair(t, carry):
        j0 = 2 * t
        j1 = j0 + 1
        wait(0, gsem_a)
        unpack(j1, 1)
        fire(1, gsem_b)
        scat(0)
        wait(1, gsem_b)
        if nchunk > PXR:
            @pl.when(j1 == PXR - 1)
            def _():
                pltpu.sync_copy(px.at[w, pl.ds(PXR, PXR)], px_v)

        @pl.when(j1 != nchunk - 1)
        def _():
            unpack(j1 + 1, 0)
            fire(0, gsem_a)

        scat(1)
        return carry

    lax.fori_loop(0, nchunk // 2, pair, 0)
    plsc.subcore_barrier()

    pltpu.sync_copy(acc_sh.at[pl.ds(s * RPT, RPT)], out.at[c, pl.ds(s * RPT, RPT)])


def _make_sc_agg(nchunk):
    mesh = plsc.VectorSubcoreMesh(core_axis_name="c", subcore_axis_name="s")
    return pl.kernel(
        functools.partial(_sc_agg_body, nchunk),
        out_type=jax.ShapeDtypeStruct((NC, NPAD, D_IN), jnp.float32),
        mesh=mesh,
        scratch_types=[
            pltpu.VMEM((PXR, CH), jnp.int32),
            pltpu.VMEM((2, CH), jnp.int32),
            pltpu.VMEM((2, CH), jnp.int32),
            pltpu.VMEM((2, CH, D_IN), jnp.float32),
            pltpu.SemaphoreType.DMA,
            pltpu.SemaphoreType.DMA,
            pltpu.VMEM_SHARED((NPAD, D_IN), jnp.float32),
        ],
    )


_sc_agg1 = _make_sc_agg(NCH1)
_sc_agg2 = _make_sc_agg(NCH_E)


BLK = 400
NBLK = N // BLK
DBLK = NPAD // 8

_dot = functools.partial(jnp.dot, preferred_element_type=jnp.float32,
                         precision=lax.Precision.HIGHEST)


def _ext_body(s3, d3, rp, ep):
    sv = s3[...]
    dv = d3[...]
    rp[...] = sv | (dv << 14)
    ep[...] = (((dv & (REP * CH - 1)) + N)
               | (((dv >> 7) + DEGBASE) << 14))


def _tc_ext(src3, dst3):
    return pl.pallas_call(
        _ext_body,
        out_shape=[
            jax.ShapeDtypeStruct((NW, NCH_E, CH), jnp.int32),
            jax.ShapeDtypeStruct((NW, NCH_E, CH), jnp.int32),
        ],
    )(src3, dst3)


def _stage_a_body(x, nk, emb, h0):
    h = x[...]
    k = nk[...].astype(jnp.float32)
    for kk in range(N_KINDS):
        mask = jnp.where(k == kk, 1.0, 0.0)
        h = h + mask * emb[kk, :][None, :]
    h0[...] = h


def _tc_stage_a(x, nk2, emb):
    return pl.pallas_call(
        _stage_a_body,
        grid=(NBLK,),
        in_specs=[
            pl.BlockSpec((BLK, D_IN), lambda i: (i, 0)),
            pl.BlockSpec((BLK, 1), lambda i: (i, 0)),
            pl.BlockSpec((N_KINDS, D_IN), lambda i: (0, 0)),
        ],
        out_specs=pl.BlockSpec((BLK, D_IN), lambda i: (i, 0)),
        out_shape=jax.ShapeDtypeStruct((N, D_IN), jnp.float32),
    )(x, nk2, emb)


def _deg_body(degp, out):
    out[...] = jnp.maximum(jnp.sum(degp[...], axis=0), 1.0)[:, None]


def _tc_deg(degp):
    return pl.pallas_call(
        _deg_body,
        grid=(8,),
        in_specs=[pl.BlockSpec((NC, DBLK), lambda i: (0, i))],
        out_specs=pl.BlockSpec((DBLK, 1), lambda i: (i, 0)),
        out_shape=jax.ShapeDtypeStruct((NPAD, 1), jnp.float32),
    )(degp)


def _stage_b_body(h0, s0a, s0b, dg, w1s, w1n, b1, w2s, w2n, z, hs):
    deg = dg[...]
    a0 = (s0a[...] + s0b[...]) / deg
    h1 = jax.nn.relu(_dot(h0[...], w1s[...]) + _dot(a0, w1n[...]) + b1[...])
    z[...] = _dot(h1, w2n[...])
    hs[...] = _dot(h1, w2s[...])


def _tc_stage_b(h0, s0a, s0b, dg, W1s, W1n, b1, W2s, W2n):
    return pl.pallas_call(
        _stage_b_body,
        grid=(NBLK,),
        in_specs=[
            pl.BlockSpec((BLK, D_IN), lambda i: (i, 0)),
            pl.BlockSpec((BLK, D_IN), lambda i: (i, 0)),
            pl.BlockSpec((BLK, D_IN), lambda i: (i, 0)),
            pl.BlockSpec((BLK, 1), lambda i: (i, 0)),
            pl.BlockSpec((D_IN, D_H), lambda i: (0, 0)),
            pl.BlockSpec((D_IN, D_H), lambda i: (0, 0)),
            pl.BlockSpec((1, D_H), lambda i: (0, 0)),
            pl.BlockSpec((D_H, D_OUT), lambda i: (0, 0)),
            pl.BlockSpec((D_H, D_OUT), lambda i: (0, 0)),
        ],
        out_specs=[
            pl.BlockSpec((BLK, D_OUT), lambda i: (i, 0)),
            pl.BlockSpec((BLK, D_OUT), lambda i: (i, 0)),
        ],
        out_shape=[
            jax.ShapeDtypeStruct((N, D_OUT), jnp.float32),
            jax.ShapeDtypeStruct((N, D_OUT), jnp.float32),
        ],
    )(h0, s0a, s0b, dg, W1s, W1n, b1, W2s, W2n)


def _stage_c_body(hs, s1a, s1b, dg, b2, wp1, bp1, wp2, bp2, p):
    deg = dg[...]
    a1 = (s1a[...] + s1b[...]) / deg
    h2 = hs[...] + a1 + b2[...]
    g = jax.nn.relu(_dot(h2, wp1[...]) + bp1[...])
    p[...] = _dot(g, wp2[...]) + bp2[...]


def _tc_stage_c(hs, s1a, s1b, dg, b2, Wp1, bp1, Wp2, bp2):
    return pl.pallas_call(
        _stage_c_body,
        grid=(NBLK,),
        in_specs=[
            pl.BlockSpec((BLK, D_OUT), lambda i: (i, 0)),
            pl.BlockSpec((BLK, D_OUT), lambda i: (i, 0)),
            pl.BlockSpec((BLK, D_OUT), lambda i: (i, 0)),
            pl.BlockSpec((BLK, 1), lambda i: (i, 0)),
            pl.BlockSpec((1, D_OUT), lambda i: (0, 0)),
            pl.BlockSpec((D_OUT, D_PRED), lambda i: (0, 0)),
            pl.BlockSpec((1, D_PRED), lambda i: (0, 0)),
            pl.BlockSpec((D_PRED, D_OUT), lambda i: (0, 0)),
            pl.BlockSpec((1, D_OUT), lambda i: (0, 0)),
        ],
        out_specs=pl.BlockSpec((BLK, D_OUT), lambda i: (i, 0)),
        out_shape=jax.ShapeDtypeStruct((N, D_OUT), jnp.float32),
    )(hs, s1a, s1b, dg, b2, Wp1, bp1, Wp2, bp2)


def kernel(x, edge_index, node_kind, family_ids, kind_emb,
           W1s, W1n, b1, W2s, W2n, b2, Wp1, bp1, Wp2, bp2):
    src = edge_index[0]
    dst = edge_index[1]
    pad = EPAD - E
    src3 = jnp.concatenate([src, jnp.zeros((pad,), jnp.int32)]).reshape(NW, NCH_E, CH)
    dst3 = jnp.concatenate([dst, jnp.full((pad,), N, jnp.int32)]).reshape(NW, NCH_E, CH)
    rp, ep = _tc_ext(src3, dst3)
    px1 = jnp.stack([rp, ep], axis=2).reshape(NW, NCH1, CH)

    zeros = jnp.zeros((NPAD, D_IN), jnp.float32)
    eye_rep = jnp.tile(jnp.eye(CH, dtype=jnp.float32), (REP, 1))

    nk2 = node_kind[:, None]
    b1r = b1[None, :]
    b2r = b2[None, :]
    bp1r = bp1[None, :]
    bp2r = bp2[None, :]

    h0 = _tc_stage_a(x, nk2, kind_emb)
    table1 = jnp.concatenate([h0, eye_rep])
    s0 = _sc_agg1(table1, px1, zeros)
    dg = _tc_deg(s0[:, DEGBASE:DEGBASE + NPAD // CH, :].reshape(NC, NPAD))
    z, hs = _tc_stage_b(h0, s0[0], s0[1], dg, W1s, W1n, b1r, W2s, W2n)
    s1 = _sc_agg2(z, rp, zeros)
    p = _tc_stage_c(hs, s1[0], s1[1], dg, b2r, Wp1, bp1r, Wp2, bp2r)
    return p
